# baseline (speedup 1.0000x reference)
import functools

import jax
import jax.numpy as jnp
from jax import lax
from jax.experimental import pallas as pl
from jax.experimental.pallas import tpu as pltpu
from jax.experimental.pallas import tpu_sc as plsc

B = 16384
D = 128
CONTENT = 384

_NC = 2
_NS = 16
_NW = _NC * _NS
_B_PER_W = B // _NW


def _sc_gather(user_table, item_table, user_id, item_id):
    mesh = plsc.VectorSubcoreMesh(core_axis_name="c", subcore_axis_name="s")
    n = user_id.shape[0]
    b_per_w = n // _NW

    @functools.partial(
        pl.kernel,
        mesh=mesh,
        out_type=(
            jax.ShapeDtypeStruct((n, D), jnp.float32),
            jax.ShapeDtypeStruct((n, D), jnp.float32),
        ),
        scratch_types=[
            pltpu.VMEM((b_per_w,), jnp.int32),
            pltpu.VMEM((b_per_w, D), jnp.float32),
            pltpu.SemaphoreType.DMA,
        ],
    )
    def k(ut_hbm, it_hbm, uid_hbm, iid_hbm, uo_hbm, io_hbm, idx_v, rows_v, sem):
        wid = lax.axis_index("s") * _NC + lax.axis_index("c")
        base = wid * b_per_w
        pltpu.sync_copy(uid_hbm.at[pl.ds(base, b_per_w)], idx_v)
        pltpu.async_copy(ut_hbm.at[idx_v], rows_v, sem).wait()
        pltpu.sync_copy(rows_v, uo_hbm.at[pl.ds(base, b_per_w)])
        pltpu.sync_copy(iid_hbm.at[pl.ds(base, b_per_w)], idx_v)
        pltpu.async_copy(it_hbm.at[idx_v], rows_v, sem).wait()
        pltpu.sync_copy(rows_v, io_hbm.at[pl.ds(base, b_per_w)])

    return k(user_table, item_table, user_id, item_id)


def _dot3(x, w):
    return jnp.dot(x, w, preferred_element_type=jnp.float32)


def _towers_body(u_ref, it_ref, c_ref, wu1_ref, bu1_ref, wu2_ref, bu2_ref,
                 wi1a_ref, wi1b_ref, bi1_ref, wi2_ref, bi2_ref, t_ref, o_ref):
    hu = _dot3(u_ref[...], wu1_ref[...])
    hu = jnp.maximum(hu + bu1_ref[...], 0.0)
    uv = _dot3(hu, wu2_ref[...]) + bu2_ref[...]
    uv = uv * lax.rsqrt(jnp.maximum(jnp.sum(uv * uv, axis=1, keepdims=True), 1e-12))
    hi = _dot3(it_ref[...], wi1a_ref[...]) + _dot3(c_ref[...], wi1b_ref[...])
    hi = jnp.maximum(hi + bi1_ref[...], 0.0)
    iv2 = _dot3(hi, wi2_ref[...]) + bi2_ref[...]
    iv2 = iv2 * lax.rsqrt(jnp.maximum(jnp.sum(iv2 * iv2, axis=1, keepdims=True), 1e-12))
    sim = jnp.sum(uv * iv2, axis=1, keepdims=True)
    o_ref[...] = jax.nn.sigmoid(sim / t_ref[0, 0])


def _towers(u_rows, i_rows, content, Wu1, bu1, Wu2, bu2, Wi1a, Wi1b, bi1,
            Wi2, bi2, temperature, bm=2048, interpret=False):
    n = u_rows.shape[0]
    grid = (n // bm,)
    row = lambda i: (i, 0)
    full = lambda i: (0, 0)
    out = pl.pallas_call(
        _towers_body,
        grid=grid,
        in_specs=[
            pl.BlockSpec((bm, D), row),
            pl.BlockSpec((bm, D), row),
            pl.BlockSpec((bm, CONTENT), row),
            pl.BlockSpec((D, 128), full),
            pl.BlockSpec((1, 128), full),
            pl.BlockSpec((128, D), full),
            pl.BlockSpec((1, D), full),
            pl.BlockSpec((D, 256), full),
            pl.BlockSpec((CONTENT, 256), full),
            pl.BlockSpec((1, 256), full),
            pl.BlockSpec((256, D), full),
            pl.BlockSpec((1, D), full),
            pl.BlockSpec((1, 1), full),
        ],
        out_specs=pl.BlockSpec((bm, 1), row),
        out_shape=jax.ShapeDtypeStruct((n, 1), jnp.float32),
        compiler_params=pltpu.CompilerParams(
            dimension_semantics=("parallel",)),
        interpret=interpret,
    )(u_rows, i_rows, content, Wu1, bu1, Wu2, bu2, Wi1a, Wi1b, bi1,
      Wi2, bi2, temperature)
    return out


@jax.jit
def kernel(user_id, item_id, content_embedding, user_table, item_table,
           Wu1, bu1, Wu2, bu2, Wi1, bi1, Wi2, bi2, temperature):
    uid = jnp.asarray(user_id, jnp.int32)
    iid = jnp.asarray(item_id, jnp.int32)
    h = B // 2
    outs = []
    for c in range(2):
        sl = slice(c * h, (c + 1) * h)
        u_rows, i_rows = _sc_gather(user_table, item_table, uid[sl], iid[sl])
        outs.append(_towers(
            u_rows, i_rows, content_embedding[sl],
            Wu1, bu1.reshape(1, -1), Wu2, bu2.reshape(1, -1),
            Wi1[:D], Wi1[D:], bi1.reshape(1, -1), Wi2, bi2.reshape(1, -1),
            temperature.reshape(1, 1),
        ))
    return jnp.concatenate(outs, axis=0)


# speedup vs baseline: 1.4259x; 1.4259x over previous
import functools

import jax
import jax.numpy as jnp
from jax import lax
from jax.experimental import pallas as pl
from jax.experimental.pallas import tpu as pltpu
from jax.experimental.pallas import tpu_sc as plsc

B = 16384
D = 128
CONTENT = 384

_NC = 2
_NS = 16
_NW = _NC * _NS
_B_PER_W = B // _NW


def _sc_gather(user_table, item_table, user_id, item_id):
    mesh = plsc.VectorSubcoreMesh(core_axis_name="c", subcore_axis_name="s")
    n = user_id.shape[0]
    b_per_w = n // _NW

    @functools.partial(
        pl.kernel,
        mesh=mesh,
        out_type=(
            jax.ShapeDtypeStruct((n, D), jnp.float32),
            jax.ShapeDtypeStruct((n, D), jnp.float32),
        ),
        scratch_types=[
            pltpu.VMEM((b_per_w,), jnp.int32),
            pltpu.VMEM((b_per_w, D), jnp.float32),
            pltpu.SemaphoreType.DMA,
        ],
    )
    def k(ut_hbm, it_hbm, uid_hbm, iid_hbm, uo_hbm, io_hbm, idx_v, rows_v, sem):
        wid = lax.axis_index("s") * _NC + lax.axis_index("c")
        base = wid * b_per_w
        pltpu.sync_copy(uid_hbm.at[pl.ds(base, b_per_w)], idx_v)
        pltpu.async_copy(ut_hbm.at[idx_v], rows_v, sem).wait()
        pltpu.sync_copy(rows_v, uo_hbm.at[pl.ds(base, b_per_w)])
        pltpu.sync_copy(iid_hbm.at[pl.ds(base, b_per_w)], idx_v)
        pltpu.async_copy(it_hbm.at[idx_v], rows_v, sem).wait()
        pltpu.sync_copy(rows_v, io_hbm.at[pl.ds(base, b_per_w)])

    return k(user_table, item_table, user_id, item_id)


def _dot3(x, w):
    return jnp.dot(x, w, preferred_element_type=jnp.float32)


def _towers_body(u_ref, it_ref, c_ref, wu1_ref, bu1_ref, wu2_ref, bu2_ref,
                 wi1a_ref, wi1b_ref, bi1_ref, wi2_ref, bi2_ref, t_ref, o_ref):
    hu = _dot3(u_ref[...], wu1_ref[...])
    hu = jnp.maximum(hu + bu1_ref[...], 0.0)
    uv = _dot3(hu, wu2_ref[...]) + bu2_ref[...]
    uv = uv * lax.rsqrt(jnp.maximum(jnp.sum(uv * uv, axis=1, keepdims=True), 1e-12))
    hi = _dot3(it_ref[...], wi1a_ref[...]) + _dot3(c_ref[...], wi1b_ref[...])
    hi = jnp.maximum(hi + bi1_ref[...], 0.0)
    iv2 = _dot3(hi, wi2_ref[...]) + bi2_ref[...]
    iv2 = iv2 * lax.rsqrt(jnp.maximum(jnp.sum(iv2 * iv2, axis=1, keepdims=True), 1e-12))
    prod_t = jnp.transpose(uv * iv2)
    sim = jnp.sum(prod_t, axis=0, keepdims=True)
    o_ref[...] = jax.nn.sigmoid(sim / t_ref[0, 0])


def _towers(u_rows, i_rows, content, Wu1, bu1, Wu2, bu2, Wi1a, Wi1b, bi1,
            Wi2, bi2, temperature, bm=4096, interpret=False):
    n = u_rows.shape[0]
    grid = (n // bm,)
    row = lambda i: (i, 0)
    full = lambda i: (0, 0)
    out = pl.pallas_call(
        _towers_body,
        grid=grid,
        in_specs=[
            pl.BlockSpec((bm, D), row),
            pl.BlockSpec((bm, D), row),
            pl.BlockSpec((bm, CONTENT), row),
            pl.BlockSpec((D, 128), full),
            pl.BlockSpec((1, 128), full),
            pl.BlockSpec((128, D), full),
            pl.BlockSpec((1, D), full),
            pl.BlockSpec((D, 256), full),
            pl.BlockSpec((CONTENT, 256), full),
            pl.BlockSpec((1, 256), full),
            pl.BlockSpec((256, D), full),
            pl.BlockSpec((1, D), full),
            pl.BlockSpec((1, 1), full),
        ],
        out_specs=pl.BlockSpec((1, bm), lambda i: (0, i)),
        out_shape=jax.ShapeDtypeStruct((1, n), jnp.float32),
        compiler_params=pltpu.CompilerParams(
            dimension_semantics=("parallel",)),
        interpret=interpret,
    )(u_rows, i_rows, content, Wu1, bu1, Wu2, bu2, Wi1a, Wi1b, bi1,
      Wi2, bi2, temperature)
    return out.reshape(n, 1)


@jax.jit
def kernel(user_id, item_id, content_embedding, user_table, item_table,
           Wu1, bu1, Wu2, bu2, Wi1, bi1, Wi2, bi2, temperature):
    uid = jnp.asarray(user_id, jnp.int32)
    iid = jnp.asarray(item_id, jnp.int32)
    u_rows, i_rows = _sc_gather(user_table, item_table, uid, iid)
    return _towers(
        u_rows, i_rows, content_embedding,
        Wu1, bu1.reshape(1, -1), Wu2, bu2.reshape(1, -1),
        Wi1[:D], Wi1[D:], bi1.reshape(1, -1), Wi2, bi2.reshape(1, -1),
        temperature.reshape(1, 1),
    )
